# baseline (speedup 1.0000x reference)
import jax
import jax.numpy as jnp
from jax import lax
from jax.experimental import pallas as pl
from jax.experimental.pallas import tpu as pltpu
from jax.experimental.pallas import tpu_sc as plsc

VOCAB = 1000000
BATCH = 4096
SEQ = 200
CH0 = 104
CH1 = SEQ - CH0
EMBED = 64
NUM_WORKERS = 32
ROWS_PER_W = BATCH // NUM_WORKERS
LANES = 16
D_CH = EMBED // LANES
PIPE = 4
N_CNT = SEQ // LANES + 1

CHUNK = 400
N_CHUNKS = VOCAB // CHUNK
STEPS = (N_CHUNKS + 2 * NUM_WORKERS - 1) // (2 * NUM_WORKERS)


def _linearize_body(w_hbm, out_hbm, buf_a, buf_b, flat_v, sem_a, sem_b):
    wid = lax.axis_index("s") * 2 + lax.axis_index("c")

    def issue(cid, buf, sem):
        @pl.when(cid < N_CHUNKS)
        def _():
            pltpu.async_copy(w_hbm.at[pl.ds(cid * CHUNK, CHUNK)], buf, sem)

    def compact_out(cid, buf, sem):
        @pl.when(cid < N_CHUNKS)
        def _():
            pltpu.make_async_copy(w_hbm.at[pl.ds(0, CHUNK)], buf, sem).wait()

            def body(i, carry):
                for d in range(D_CH):
                    flat_v[pl.ds(i * EMBED + d * LANES, LANES)] = (
                        buf[i, pl.ds(d * LANES, LANES)])
                return carry

            lax.fori_loop(0, CHUNK, body, 0, unroll=8)
            pltpu.sync_copy(
                flat_v, out_hbm.at[pl.ds(cid * CHUNK * EMBED, CHUNK * EMBED)])

    issue(wid, buf_a, sem_a)

    def step(j, carry):
        cid0 = (2 * j) * NUM_WORKERS + wid
        cid1 = (2 * j + 1) * NUM_WORKERS + wid
        issue(cid1, buf_b, sem_b)
        compact_out(cid0, buf_a, sem_a)
        issue(cid1 + NUM_WORKERS, buf_a, sem_a)
        compact_out(cid1, buf_b, sem_b)
        return carry

    lax.fori_loop(0, STEPS, step, 0)


def _gather_body(idx_hbm, w_hbm, out_hbm, idx_v, rows_v, out_v, *sems):
    wid = lax.axis_index("s") * 2 + lax.axis_index("c")
    base = wid * ROWS_PER_W
    pltpu.sync_copy(idx_hbm.at[pl.ds(base, ROWS_PER_W)], idx_v)

    def issue(row, j):
        pltpu.async_copy(w_hbm.at[idx_v.at[row, pl.ds(0, CH0)]],
                         rows_v.at[j, pl.ds(0, CH0)], sems[j])
        pltpu.async_copy(w_hbm.at[idx_v.at[row, pl.ds(CH0, CH1)]],
                         rows_v.at[j, pl.ds(CH0, CH1)], sems[j])

    def drain(j):
        pltpu.make_async_copy(w_hbm.at[pl.ds(0, SEQ)],
                              rows_v.at[j], sems[j]).wait()

    lane = lax.iota(jnp.int32, LANES)

    def accumulate(row, j):
        cnt = jnp.zeros((LANES,), jnp.int32)
        for c in range(N_CNT):
            off = min(c * LANES, SEQ - LANES)
            iv = idx_v[row, pl.ds(off, LANES)]
            nz = iv != 0
            if c * LANES > off:
                nz = jnp.logical_and(nz, lane >= (c * LANES - off))
            cnt = cnt + plsc.all_reduce_population_count(nz)
        inv = 1.0 / jnp.maximum(cnt.astype(jnp.float32), 1.0)

        def inner(i, accs):
            return tuple(accs[d] + rows_v[j, i, pl.ds(d * LANES, LANES)]
                         for d in range(D_CH))

        zeros = tuple(jnp.zeros((LANES,), jnp.float32) for _ in range(D_CH))
        accs = lax.fori_loop(0, SEQ, inner, zeros, unroll=8)
        for d in range(D_CH):
            out_v[row, pl.ds(d * LANES, LANES)] = accs[d] * inv

    for j in range(PIPE):
        issue(j, j)

    def block_body(k, carry):
        for j in range(PIPE):
            row = k * PIPE + j
            drain(j)
            accumulate(row, j)

            @pl.when(row + PIPE < ROWS_PER_W)
            def _():
                issue(row + PIPE, j)
        return carry

    lax.fori_loop(0, ROWS_PER_W // PIPE, block_body, 0)
    pltpu.sync_copy(out_v, out_hbm.at[pl.ds(base, ROWS_PER_W)])


def kernel(sentence, weight):
    idx = sentence.astype(jnp.int32)

    linearize = pl.kernel(
        _linearize_body,
        out_type=jax.ShapeDtypeStruct((VOCAB * EMBED,), jnp.float32),
        mesh=plsc.VectorSubcoreMesh(core_axis_name="c", subcore_axis_name="s"),
        scratch_types=[
            pltpu.VMEM((CHUNK, EMBED), jnp.float32),
            pltpu.VMEM((CHUNK, EMBED), jnp.float32),
            pltpu.VMEM((CHUNK * EMBED,), jnp.float32),
            pltpu.SemaphoreType.DMA,
            pltpu.SemaphoreType.DMA,
        ],
        compiler_params=pltpu.CompilerParams(use_tc_tiling_on_sc=True,
                                             needs_layout_passes=False),
    )
    w_lin = linearize(weight).reshape(VOCAB, EMBED)

    gather = pl.kernel(
        _gather_body,
        out_type=jax.ShapeDtypeStruct((BATCH, EMBED), jnp.float32),
        mesh=plsc.VectorSubcoreMesh(core_axis_name="c", subcore_axis_name="s"),
        scratch_types=[
            pltpu.VMEM((ROWS_PER_W, SEQ), jnp.int32),
            pltpu.VMEM((PIPE, SEQ, EMBED), jnp.float32),
            pltpu.VMEM((ROWS_PER_W, EMBED), jnp.float32),
        ] + [pltpu.SemaphoreType.DMA] * PIPE,
        compiler_params=pltpu.CompilerParams(use_tc_tiling_on_sc=False,
                                             needs_layout_passes=False),
    )
    return gather(idx, w_lin)


# speedup vs baseline: 1.3768x; 1.3768x over previous
import jax
import jax.numpy as jnp
from jax import lax
from jax.experimental import pallas as pl
from jax.experimental.pallas import tpu as pltpu
from jax.experimental.pallas import tpu_sc as plsc

BATCH = 4096
SEQ = 200
CH0 = 104
CH1 = SEQ - CH0
EMBED = 64
NUM_WORKERS = 32
ROWS_PER_W = BATCH // NUM_WORKERS
LANES = 16
D_CH = EMBED // LANES
PIPE = 4
N_CNT = SEQ // LANES + 1


def _body(idx_hbm, w_hbm, out_hbm, idx_v, rows_v, out_v, *sems):
    wid = lax.axis_index("s") * 2 + lax.axis_index("c")
    base = wid * ROWS_PER_W
    pltpu.sync_copy(idx_hbm.at[pl.ds(base, ROWS_PER_W)], idx_v)

    def issue(row, j):
        pltpu.async_copy(w_hbm.at[idx_v.at[row, pl.ds(0, CH0)]],
                         rows_v.at[j, pl.ds(0, CH0)], sems[j])
        pltpu.async_copy(w_hbm.at[idx_v.at[row, pl.ds(CH0, CH1)]],
                         rows_v.at[j, pl.ds(CH0, CH1)], sems[j])

    def drain(j):
        pltpu.make_async_copy(w_hbm.at[pl.ds(0, SEQ)],
                              rows_v.at[j], sems[j]).wait()

    lane = lax.iota(jnp.int32, LANES)

    def accumulate(row, j):
        cnt = jnp.zeros((LANES,), jnp.int32)
        for c in range(N_CNT):
            off = min(c * LANES, SEQ - LANES)
            iv = idx_v[row, pl.ds(off, LANES)]
            nz = iv != 0
            if c * LANES > off:
                nz = jnp.logical_and(nz, lane >= (c * LANES - off))
            cnt = cnt + plsc.all_reduce_population_count(nz)
        inv = 1.0 / jnp.maximum(cnt.astype(jnp.float32), 1.0)

        def inner(i, accs):
            return tuple(accs[d] + rows_v[j, i, pl.ds(d * LANES, LANES)]
                         for d in range(D_CH))

        zeros = tuple(jnp.zeros((LANES,), jnp.float32) for _ in range(D_CH))
        accs = lax.fori_loop(0, SEQ, inner, zeros, unroll=8)
        for d in range(D_CH):
            out_v[row, pl.ds(d * LANES, LANES)] = accs[d] * inv

    for j in range(PIPE):
        issue(j, j)

    def block_body(k, carry):
        for j in range(PIPE):
            row = k * PIPE + j
            drain(j)
            accumulate(row, j)

            @pl.when(row + PIPE < ROWS_PER_W)
            def _():
                issue(row + PIPE, j)
        return carry

    lax.fori_loop(0, ROWS_PER_W // PIPE, block_body, 0)
    pltpu.sync_copy(out_v, out_hbm.at[pl.ds(base, ROWS_PER_W)])


def kernel(sentence, weight):
    idx = sentence.astype(jnp.int32)
    f = pl.kernel(
        _body,
        out_type=jax.ShapeDtypeStruct((BATCH, EMBED), jnp.float32),
        mesh=plsc.VectorSubcoreMesh(core_axis_name="c", subcore_axis_name="s"),
        scratch_types=[
            pltpu.VMEM((ROWS_PER_W, SEQ), jnp.int32),
            pltpu.VMEM((PIPE, SEQ, EMBED), jnp.float32),
            pltpu.VMEM((ROWS_PER_W, EMBED), jnp.float32),
        ] + [pltpu.SemaphoreType.DMA] * PIPE,
        compiler_params=pltpu.CompilerParams(use_tc_tiling_on_sc=False,
                                             needs_layout_passes=False),
    )
    return f(idx, weight)
